# baseline (speedup 1.0000x reference)
import jax
import jax.numpy as jnp
from jax.experimental import pallas as pl
from jax.experimental.pallas import tpu as pltpu

_B, _P, _C = 64, 8732, 21
_N = _B * _P
_W = 9472
_G = _N // _W


def _csd_block(conf_ref, conff_ref, loc_ref, locf_ref, out_ref, acc_ref):
    i = pl.program_id(0)

    @pl.when(i == 0)
    def _init():
        acc_ref[0] = 0.0
        acc_ref[1] = 0.0
        acc_ref[2] = 0.0

    c = conf_ref[...]
    cf = conff_ref[...]
    p = c + 1e-7
    q = cf + 1e-7
    kl = (q - p) * jnp.log(q / p)
    row = jax.lax.broadcasted_iota(jnp.int32, c.shape, 0)
    fg = jnp.max(jnp.where(row >= 1, c, -1e30), axis=0, keepdims=True)
    bg = jnp.max(jnp.where(row == 0, c, -1e30), axis=0, keepdims=True)
    mask = (fg > bg).astype(jnp.float32)
    klrow = jnp.sum(kl, axis=0, keepdims=True)

    l = loc_ref[...]
    lf = locf_ref[...]
    lrow = jax.lax.broadcasted_iota(jnp.int32, l.shape, 0)
    t = jnp.where(lrow == 0, l + lf, l - lf)
    t2 = jnp.sum(t * t, axis=0, keepdims=True)

    acc_ref[0] += jnp.sum(mask * klrow)
    acc_ref[1] += jnp.sum(mask * t2)
    acc_ref[2] += jnp.sum(mask)

    @pl.when(i == _G - 1)
    def _fin():
        cnt = jnp.maximum(acc_ref[2], 1.0)
        out_ref[0, 0] = acc_ref[0] / (2.0 * cnt) + acc_ref[1] / (4.0 * cnt)


def kernel(conf, conf_flip, loc, loc_flip):
    ct = conf.transpose(2, 0, 1).reshape(_C, _N)
    cft = conf_flip.transpose(2, 0, 1).reshape(_C, _N)
    lt = loc.transpose(2, 0, 1).reshape(4, _N)
    lft = loc_flip.transpose(2, 0, 1).reshape(4, _N)
    out = pl.pallas_call(
        _csd_block,
        grid=(_G,),
        in_specs=[
            pl.BlockSpec((_C, _W), lambda i: (0, i)),
            pl.BlockSpec((_C, _W), lambda i: (0, i)),
            pl.BlockSpec((4, _W), lambda i: (0, i)),
            pl.BlockSpec((4, _W), lambda i: (0, i)),
        ],
        out_specs=pl.BlockSpec(memory_space=pltpu.SMEM),
        out_shape=jax.ShapeDtypeStruct((1, 1), jnp.float32),
        scratch_shapes=[pltpu.SMEM((3,), jnp.float32)],
    )(ct, cft, lt, lft)
    return out[0, 0]
